# Initial kernel scaffold; baseline (speedup 1.0000x reference)
import jax
import jax.numpy as jnp
from jax.experimental import pallas as pl


def kernel(x, relative_pe):
    raise NotImplementedError("write your pallas kernel here")


# speedup vs baseline: 32.1398x; 32.1398x over previous
import jax
import jax.numpy as jnp
from jax import lax
from jax.experimental import pallas as pl
from jax.experimental.pallas import tpu as pltpu
from jax.experimental.pallas import tpu_sc as plsc

_MAX_REL = 128
_NUM_PE = 2 * _MAX_REL + 1
_S = 512
_TBL = 2 * _S
_D = 1024

_NC, _NS = 2, 16
_NW = _NC * _NS
_RPW = _S // _NW
_LANES = 16
_CPR = _D // _LANES


def _prefix_body(pe_ref, t_ref):
    u = lax.broadcasted_iota(jnp.int32, (_TBL, _NUM_PE), 0)
    k = lax.broadcasted_iota(jnp.int32, (_TBL, _NUM_PE), 1)
    first = jnp.minimum(u, _S - _MAX_REL).astype(jnp.float32)
    last = jnp.maximum(u - (_S + _MAX_REL - 1), 0).astype(jnp.float32)
    interior = (u >= k + (_S - _MAX_REL)).astype(jnp.float32)
    m = jnp.where(k == 0, first, jnp.where(k == _NUM_PE - 1, last, interior))
    m = m * (1.0 / _S)
    t_ref[...] = jnp.dot(m, pe_ref[...], preferred_element_type=jnp.float32,
                         precision=lax.Precision.HIGHEST)


def _sc_body(t_hbm, out_hbm, hi_v, lo_v, res_v, sem_hi, sem_lo):
    wid = lax.axis_index("s") * _NC + lax.axis_index("c")
    base = wid * _RPW
    r16 = lax.iota(jnp.int32, _LANES)
    idx_hi = (_TBL - 1 - base) - r16
    idx_lo = (_S - 1 - base) - r16
    cp_hi = pltpu.async_copy(t_hbm.at[idx_hi], hi_v, sem_hi)
    cp_lo = pltpu.async_copy(t_hbm.at[idx_lo], lo_v, sem_lo)
    cp_hi.wait()
    cp_lo.wait()

    def row(r, carry):
        for c in range(_CPR):
            sl = pl.ds(c * _LANES, _LANES)
            res_v[r, sl] = hi_v[r, sl] - lo_v[r, sl]
        return carry

    lax.fori_loop(0, _RPW, row, 0)
    pltpu.sync_copy(res_v, out_hbm.at[pl.ds(base, _RPW)])


_sc_gather = pl.kernel(
    _sc_body,
    out_type=jax.ShapeDtypeStruct((_S, _D), jnp.float32),
    mesh=plsc.VectorSubcoreMesh(core_axis_name="c", subcore_axis_name="s"),
    scratch_types=[
        pltpu.VMEM((_RPW, _D), jnp.float32),
        pltpu.VMEM((_RPW, _D), jnp.float32),
        pltpu.VMEM((_RPW, _D), jnp.float32),
        pltpu.SemaphoreType.DMA,
        pltpu.SemaphoreType.DMA,
    ],
)


def _add_body(x_ref, pe_ref, o_ref):
    o_ref[...] = x_ref[...] + pe_ref[...][None, :, :]


def kernel(x, relative_pe):
    t = pl.pallas_call(
        _prefix_body,
        out_shape=jax.ShapeDtypeStruct((_TBL, _D), jnp.float32),
    )(relative_pe)
    pe_enc = _sc_gather(t)
    out = pl.pallas_call(
        _add_body,
        grid=(x.shape[0],),
        in_specs=[
            pl.BlockSpec((1, _S, _D), lambda b: (b, 0, 0)),
            pl.BlockSpec((_S, _D), lambda b: (0, 0)),
        ],
        out_specs=pl.BlockSpec((1, _S, _D), lambda b: (b, 0, 0)),
        out_shape=jax.ShapeDtypeStruct(x.shape, x.dtype),
    )(x, pe_enc)
    return out


# speedup vs baseline: 37.3637x; 1.1625x over previous
import jax
import jax.numpy as jnp
from jax import lax
from jax.experimental import pallas as pl
from jax.experimental.pallas import tpu as pltpu
from jax.experimental.pallas import tpu_sc as plsc

_MAX_REL = 128
_NUM_PE = 2 * _MAX_REL + 1
_S = 512
_TBL = 2 * _S
_D = 1024

_NC, _NS = 2, 16
_NW = _NC * _NS
_RPW = _S // _NW
_LANES = 16
_CPR = _D // _LANES


def _prefix_body(pe_ref, d_ref):
    u = lax.broadcasted_iota(jnp.int32, (_S, _NUM_PE), 0)
    k = lax.broadcasted_iota(jnp.int32, (_S, _NUM_PE), 1)
    first = jnp.maximum((_S - _MAX_REL) - u, 0).astype(jnp.float32)
    last = jnp.maximum(u - (_MAX_REL - 1), 0).astype(jnp.float32)
    interior = ((u >= k - _MAX_REL) & (u <= k + (_S - _MAX_REL - 1))).astype(jnp.float32)
    w = jnp.where(k == 0, first, jnp.where(k == _NUM_PE - 1, last, interior))
    w = w * (1.0 / _S)
    d_ref[...] = jnp.dot(w, pe_ref[...], preferred_element_type=jnp.float32,
                         precision=lax.Precision.HIGHEST)


def _sc_body(d_hbm, out_hbm, rows_v, sem):
    wid = lax.axis_index("s") * _NC + lax.axis_index("c")
    base = wid * _RPW
    r16 = lax.iota(jnp.int32, _LANES)
    idx = (_S - 1 - base) - r16
    pltpu.async_copy(d_hbm.at[idx], rows_v, sem).wait()
    pltpu.sync_copy(rows_v, out_hbm.at[pl.ds(base, _RPW)])


_sc_gather = pl.kernel(
    _sc_body,
    out_type=jax.ShapeDtypeStruct((_S, _D), jnp.float32),
    mesh=plsc.VectorSubcoreMesh(core_axis_name="c", subcore_axis_name="s"),
    scratch_types=[
        pltpu.VMEM((_RPW, _D), jnp.float32),
        pltpu.SemaphoreType.DMA,
    ],
)


def _add_body(x_ref, pe_ref, o_ref):
    o_ref[...] = x_ref[...] + pe_ref[...][None, :, :]


def kernel(x, relative_pe):
    d = pl.pallas_call(
        _prefix_body,
        out_shape=jax.ShapeDtypeStruct((_S, _D), jnp.float32),
    )(relative_pe)
    pe_enc = _sc_gather(d)
    out = pl.pallas_call(
        _add_body,
        grid=(x.shape[0],),
        in_specs=[
            pl.BlockSpec((1, _S, _D), lambda b: (b, 0, 0)),
            pl.BlockSpec((_S, _D), lambda b: (0, 0)),
        ],
        out_specs=pl.BlockSpec((1, _S, _D), lambda b: (b, 0, 0)),
        out_shape=jax.ShapeDtypeStruct(x.shape, x.dtype),
    )(x, pe_enc)
    return out
